# Initial kernel scaffold; baseline (speedup 1.0000x reference)
import jax
import jax.numpy as jnp
from jax.experimental import pallas as pl


def kernel(nodes, neigh_idx, features, W):
    raise NotImplementedError("write your pallas kernel here")


# speedup vs baseline: 1.8192x; 1.8192x over previous
import functools

import jax
import jax.numpy as jnp
from jax import lax
from jax.experimental import pallas as pl
from jax.experimental.pallas import tpu as pltpu
from jax.experimental.pallas import tpu_sc as plsc

D = 128
S = 10
NC, NS, L = 2, 16, 16
NW = NC * NS
CHUNK = 64
IPG = 128
NG = (CHUNK * S) // IPG


def _sc_gather_body(nodes_hbm, nidx_hbm, feat_hbm, selfout_hbm, nsumout_hbm,
                    nodes_v, nidx_v, selfrows_v, nrows_v, nsum_v, sem, bpw):
    wid = lax.axis_index("s") * NC + lax.axis_index("c")
    base = wid * bpw
    nchunks = bpw // CHUNK

    def chunk_body(c, _):
        row0 = base + c * CHUNK
        pltpu.sync_copy(nodes_hbm.at[pl.ds(row0, CHUNK)], nodes_v)
        for j in range(NG):
            pltpu.sync_copy(nidx_hbm.at[pl.ds(row0 * S + j * IPG, IPG)],
                            nidx_v.at[j])
        cps = []
        cps.append(pltpu.async_copy(feat_hbm.at[nodes_v], selfrows_v, sem))
        for j in range(NG):
            cps.append(pltpu.async_copy(
                feat_hbm.at[nidx_v.at[j]],
                nrows_v.at[pl.ds(j * IPG, IPG)], sem))
        for cp in cps:
            cp.wait()
        def node_body(i, _):
            r0 = i * S
            for v in range(D // L):
                sl = pl.ds(v * L, L)
                acc = nrows_v[r0, sl]
                for s in range(1, S):
                    acc = acc + nrows_v[r0 + s, sl]
                nsum_v[i, sl] = acc
            return _
        lax.fori_loop(0, CHUNK, node_body, None)
        pltpu.sync_copy(selfrows_v, selfout_hbm.at[pl.ds(row0, CHUNK)])
        pltpu.sync_copy(nsum_v, nsumout_hbm.at[pl.ds(row0, CHUNK)])
        return _

    lax.fori_loop(0, nchunks, chunk_body, None)


def _make_sc_gather(Bp):
    bpw = Bp // NW
    mesh = plsc.VectorSubcoreMesh(core_axis_name="c", subcore_axis_name="s")
    return pl.kernel(
        functools.partial(_sc_gather_body, bpw=bpw),
        out_type=(
            jax.ShapeDtypeStruct((Bp, D), jnp.float32),
            jax.ShapeDtypeStruct((Bp, D), jnp.float32),
        ),
        mesh=mesh,
        scratch_types=[
            pltpu.VMEM((CHUNK,), jnp.int32),
            pltpu.VMEM((NG, IPG), jnp.int32),
            pltpu.VMEM((CHUNK, D), jnp.float32),
            pltpu.VMEM((CHUNK * S, D), jnp.float32),
            pltpu.VMEM((CHUNK, D), jnp.float32),
            pltpu.SemaphoreType.DMA,
        ],
    )


def _tc_matmul_body(w_ref, self_ref, nsum_ref, out_ref):
    w = w_ref[...]
    s = self_ref[...]
    n = nsum_ref[...]
    d1 = lax.dot_general(w[:, :D], s, (((1,), (1,)), ((), ())),
                         preferred_element_type=jnp.float32)
    d2 = lax.dot_general(w[:, D:], n, (((1,), (1,)), ((), ())),
                         preferred_element_type=jnp.float32)
    out_ref[...] = jnp.maximum(d1 + d2 * (1.0 / S), 0.0)


def kernel(nodes, neigh_idx, features, W):
    B = nodes.shape[0]
    Bp = -(-B // (NW * CHUNK)) * (NW * CHUNK)
    nodes_p = jnp.pad(nodes, (0, Bp - B))
    nidx_p = jnp.pad(neigh_idx, ((0, Bp - B), (0, 0))).reshape(Bp * S)

    self_f, nsum = _make_sc_gather(Bp)(nodes_p, nidx_p, features)

    BBLK = 512
    ngrid = -(-B // BBLK)
    out = pl.pallas_call(
        _tc_matmul_body,
        grid=(ngrid,),
        in_specs=[
            pl.BlockSpec((128, 2 * D), lambda i: (0, 0)),
            pl.BlockSpec((BBLK, D), lambda i: (i, 0)),
            pl.BlockSpec((BBLK, D), lambda i: (i, 0)),
        ],
        out_specs=pl.BlockSpec((128, BBLK), lambda i: (0, i)),
        out_shape=jax.ShapeDtypeStruct((128, B), jnp.float32),
    )(W, self_f, nsum)
    return out


# speedup vs baseline: 2.2395x; 1.2310x over previous
import jax
import jax.numpy as jnp
from jax import lax
from jax.experimental import pallas as pl
from jax.experimental.pallas import tpu as pltpu
from jax.experimental.pallas import tpu_sc as plsc

D = 128
S = 10
RPN = S + 1
NC, NS, L = 2, 16, 16
NW = NC * NS
CHUNK = 32
CR = CHUNK * RPN
IPG = 88
NG = CR // IPG


def _sc_body(idx_hbm, table_hbm, out_hbm,
             idx_v, rows0, rows1, outc0, outc1,
             gsem0, gsem1, osem0, osem1, *, bpw, nchunks):
    wid = lax.axis_index("s") * NC + lax.axis_index("c")
    base = wid * bpw
    rows = (rows0, rows1)
    outc = (outc0, outc1)
    gsem = (gsem0, gsem1)
    osem = (osem0, osem1)

    pltpu.sync_copy(idx_hbm.at[pl.ds(base * RPN, bpw * RPN)], idx_v)

    def fire_gathers(b, c):
        for j in range(NG):
            pltpu.async_copy(
                table_hbm.at[idx_v.at[pl.ds(c * CR + j * IPG, IPG)]],
                rows[b].at[pl.ds(j * IPG, IPG)], gsem[b])

    def drain_gathers(b):
        for j in range(NG):
            pltpu.make_async_copy(
                table_hbm.at[idx_v.at[pl.ds(j * IPG, IPG)]],
                rows[b].at[pl.ds(j * IPG, IPG)], gsem[b]).wait()

    def compute(b):
        def node_body(i, carry):
            r0 = i * RPN
            for v in range(D // L):
                sl = pl.ds(v * L, L)
                acc = rows[b][r0, sl]
                for s in range(1, RPN):
                    acc = acc + rows[b][r0 + s, sl]
                outc[b][i, sl] = jnp.maximum(acc, 0.0)
            return carry
        lax.fori_loop(0, CHUNK, node_body, 0)

    def fire_out(b, c):
        pltpu.async_copy(outc[b], out_hbm.at[pl.ds(base + c * CHUNK, CHUNK)],
                         osem[b])

    def drain_out(b):
        pltpu.make_async_copy(outc[b], out_hbm.at[pl.ds(base, CHUNK)],
                              osem[b]).wait()

    fire_gathers(0, 0)

    def superstep(ss, carry):
        c0 = 2 * ss

        @pl.when(ss > 0)
        def _():
            drain_out(1)
        fire_gathers(1, c0 + 1)
        drain_gathers(0)

        @pl.when(ss > 0)
        def _():
            drain_out(0)
        compute(0)
        fire_out(0, c0)

        @pl.when(c0 + 2 < nchunks)
        def _():
            fire_gathers(0, c0 + 2)
        drain_gathers(1)
        compute(1)
        fire_out(1, c0 + 1)
        return carry

    lax.fori_loop(0, nchunks // 2, superstep, 0)
    drain_out(0)
    drain_out(1)


def _make_sc(Bp):
    bpw = Bp // NW
    nchunks = bpw // CHUNK
    assert nchunks % 2 == 0
    mesh = plsc.VectorSubcoreMesh(core_axis_name="c", subcore_axis_name="s")

    def body(idx_hbm, table_hbm, out_hbm, *scratch):
        _sc_body(idx_hbm, table_hbm, out_hbm, *scratch,
                 bpw=bpw, nchunks=nchunks)

    return pl.kernel(
        body,
        out_type=jax.ShapeDtypeStruct((Bp, D), jnp.float32),
        mesh=mesh,
        scratch_types=[
            pltpu.VMEM((bpw * RPN,), jnp.int32),
            pltpu.VMEM((CR, D), jnp.float32),
            pltpu.VMEM((CR, D), jnp.float32),
            pltpu.VMEM((CHUNK, D), jnp.float32),
            pltpu.VMEM((CHUNK, D), jnp.float32),
            pltpu.SemaphoreType.DMA,
            pltpu.SemaphoreType.DMA,
            pltpu.SemaphoreType.DMA,
            pltpu.SemaphoreType.DMA,
        ],
    )


def _table_body(w_ref, f_ref, out_ref):
    f = f_ref[...]
    w = w_ref[...]
    d1 = lax.dot_general(f, w[:, :D], (((1,), (1,)), ((), ())),
                         preferred_element_type=jnp.float32)
    d2 = lax.dot_general(f, w[:, D:], (((1,), (1,)), ((), ())),
                         preferred_element_type=jnp.float32)
    out_ref[0] = d1
    out_ref[1] = d2 * (1.0 / S)


def kernel(nodes, neigh_idx, features, W):
    B = nodes.shape[0]
    N = features.shape[0]

    FBLK = 400
    fgrid = N // FBLK
    table = pl.pallas_call(
        _table_body,
        grid=(fgrid,),
        in_specs=[
            pl.BlockSpec((128, 2 * D), lambda i: (0, 0)),
            pl.BlockSpec((FBLK, D), lambda i: (i, 0)),
        ],
        out_specs=pl.BlockSpec((2, FBLK, D), lambda i: (0, i, 0)),
        out_shape=jax.ShapeDtypeStruct((2, N, D), jnp.float32),
    )(W, features).reshape(2 * N, D)

    Bp = -(-B // (NW * 2 * CHUNK)) * (NW * 2 * CHUNK)
    nodes_p = jnp.pad(nodes, (0, Bp - B))
    nidx_p = jnp.pad(neigh_idx, ((0, Bp - B), (0, 0)))
    idx_all = jnp.concatenate([nodes_p[:, None], nidx_p + N],
                              axis=1).reshape(Bp * RPN)

    out_bt = _make_sc(Bp)(idx_all, table)
    return out_bt[:B].T


# speedup vs baseline: 5.2547x; 2.3464x over previous
import jax
import jax.numpy as jnp
from jax import lax
from jax.experimental import pallas as pl
from jax.experimental.pallas import tpu as pltpu
from jax.experimental.pallas import tpu_sc as plsc

D = 128
S = 10
RPN = S + 1
NC, NS, L = 2, 16, 16
NW = NC * NS
CHUNK = 32
CR = CHUNK * RPN
IPG = 88
NG = CR // IPG


def _sc_body(idx_hbm, table_hbm, out_hbm,
             idx_v, rows0, rows1, outc0, outc1,
             gsem0, gsem1, osem0, osem1, *, bpw, nchunks):
    wid = lax.axis_index("s") * NC + lax.axis_index("c")
    base = wid * bpw
    rows = (rows0, rows1)
    outc = (outc0, outc1)
    gsem = (gsem0, gsem1)
    osem = (osem0, osem1)

    pltpu.sync_copy(idx_hbm.at[pl.ds(base * RPN, bpw * RPN)], idx_v)

    def fire_gathers(b, c):
        for j in range(NG):
            pltpu.async_copy(
                table_hbm.at[idx_v.at[pl.ds(c * CR + j * IPG, IPG)]],
                rows[b].at[pl.ds(j * IPG, IPG)], gsem[b])

    def drain_gathers(b):
        for j in range(NG):
            pltpu.make_async_copy(
                table_hbm.at[idx_v.at[pl.ds(j * IPG, IPG)]],
                rows[b].at[pl.ds(j * IPG, IPG)], gsem[b]).wait()

    def compute(b):
        def node_body(i, carry):
            r0 = i * RPN
            for v in range(D // L):
                sl = pl.ds(v * L, L)
                acc = rows[b][r0, sl]
                for s in range(1, RPN):
                    acc = acc + rows[b][r0 + s, sl]
                outc[b][i, sl] = jnp.maximum(acc, 0.0)
            return carry
        lax.fori_loop(0, CHUNK, node_body, 0)

    def fire_out(b, c):
        pltpu.async_copy(outc[b], out_hbm.at[pl.ds(base + c * CHUNK, CHUNK)],
                         osem[b])

    def drain_out(b):
        pltpu.make_async_copy(outc[b], out_hbm.at[pl.ds(base, CHUNK)],
                              osem[b]).wait()

    fire_gathers(0, 0)

    def superstep(ss, carry):
        c0 = 2 * ss

        @pl.when(ss > 0)
        def _():
            drain_out(1)
        fire_gathers(1, c0 + 1)
        drain_gathers(0)

        @pl.when(ss > 0)
        def _():
            drain_out(0)
        compute(0)
        fire_out(0, c0)

        @pl.when(c0 + 2 < nchunks)
        def _():
            fire_gathers(0, c0 + 2)
        drain_gathers(1)
        compute(1)
        fire_out(1, c0 + 1)
        return carry

    lax.fori_loop(0, nchunks // 2, superstep, 0)
    drain_out(0)
    drain_out(1)


def _make_sc(Bp):
    bpw = Bp // NW
    nchunks = bpw // CHUNK
    assert nchunks % 2 == 0
    mesh = plsc.VectorSubcoreMesh(core_axis_name="c", subcore_axis_name="s")

    def body(idx_hbm, table_hbm, out_hbm, *scratch):
        _sc_body(idx_hbm, table_hbm, out_hbm, *scratch,
                 bpw=bpw, nchunks=nchunks)

    return pl.kernel(
        body,
        out_type=jax.ShapeDtypeStruct((Bp, D), jnp.float32),
        mesh=mesh,
        scratch_types=[
            pltpu.VMEM((bpw * RPN,), jnp.int32),
            pltpu.VMEM((CR, D), jnp.float32),
            pltpu.VMEM((CR, D), jnp.float32),
            pltpu.VMEM((CHUNK, D), jnp.float32),
            pltpu.VMEM((CHUNK, D), jnp.float32),
            pltpu.SemaphoreType.DMA,
            pltpu.SemaphoreType.DMA,
            pltpu.SemaphoreType.DMA,
            pltpu.SemaphoreType.DMA,
        ],
    )


def _table_body(w_ref, f_ref, out_ref):
    f = f_ref[...]
    w = w_ref[...]
    d1 = lax.dot_general(f, w[:, :D], (((1,), (1,)), ((), ())),
                         preferred_element_type=jnp.float32)
    d2 = lax.dot_general(f, w[:, D:], (((1,), (1,)), ((), ())),
                         preferred_element_type=jnp.float32)
    out_ref[0] = d1
    out_ref[1] = d2 * (1.0 / S)


def kernel(nodes, neigh_idx, features, W):
    B = nodes.shape[0]
    N = features.shape[0]

    FBLK = 400
    fgrid = N // FBLK
    table = pl.pallas_call(
        _table_body,
        grid=(fgrid,),
        in_specs=[
            pl.BlockSpec((128, 2 * D), lambda i: (0, 0)),
            pl.BlockSpec((FBLK, D), lambda i: (i, 0)),
        ],
        out_specs=pl.BlockSpec((2, FBLK, D), lambda i: (0, i, 0)),
        out_shape=jax.ShapeDtypeStruct((2, N, D), jnp.float32),
    )(W, features).reshape(2 * N, D)

    Bp = -(-B // (NW * 2 * CHUNK)) * (NW * 2 * CHUNK)
    npad = Bp - B
    pad_nodes = (jnp.arange(npad, dtype=nodes.dtype) * 1031) % N
    pad_neigh = ((jnp.arange(npad * S, dtype=nodes.dtype) * 523) % N
                 ).reshape(npad, S)
    nodes_p = jnp.concatenate([nodes, pad_nodes])
    nidx_p = jnp.concatenate([neigh_idx, pad_neigh], axis=0)
    idx_all = jnp.concatenate([nodes_p[:, None], nidx_p + N],
                              axis=1).reshape(Bp * RPN)

    out_bt = _make_sc(Bp)(idx_all, table)
    return out_bt[:B].T
